# baseline (speedup 1.0000x reference)
import functools

import jax
import jax.numpy as jnp
from jax import lax
from jax.experimental import pallas as pl
from jax.experimental.pallas import tpu as pltpu
from jax.experimental.pallas import tpu_sc as plsc

N = 10000
E = 160000
D = 256
H = 128
NT = 10112
TILES = 16
EPT = E // TILES
K = 128
CHUNKS = 80
PAD = CHUNKS * K - EPT
PASS = CHUNKS // 2
RPT = NT // TILES
LASTV = N - (TILES - 1) * RPT
R = 400
NRB = N // R

_mesh = plsc.VectorSubcoreMesh(core_axis_name="c", subcore_axis_name="s")


@functools.partial(
    pl.kernel,
    out_type=jax.ShapeDtypeStruct((2, NT, H), jnp.float32),
    mesh=_mesh,
    scratch_types=[
        pltpu.VMEM((CHUNKS, K), jnp.int32),
        pltpu.VMEM((K, H), jnp.float32),
        pltpu.VMEM_SHARED((NT, H), jnp.float32),
    ],
)
def _deg_kernel(col3_hbm, ones_hbm, zeros_hbm, out_hbm, colv, onesv, table):
    c = lax.axis_index("c")
    s = lax.axis_index("s")
    r0 = s * RPT
    pltpu.sync_copy(zeros_hbm.at[pl.ds(r0, RPT)], table.at[pl.ds(r0, RPT)])
    pltpu.sync_copy(ones_hbm, onesv)
    pltpu.sync_copy(col3_hbm.at[s], colv)
    plsc.subcore_barrier()
    k0 = c * PASS

    def body(k, carry):
        pltpu.sync_copy(onesv, table.at[colv.at[k0 + k]], add=True)
        return carry

    lax.fori_loop(0, PASS, body, 0)
    plsc.subcore_barrier()
    pltpu.sync_copy(table.at[pl.ds(r0, RPT)], out_hbm.at[c, pl.ds(r0, RPT)])


def _mm_body(x_ref, w_ref, p_ref, y3_ref, dinv_ref):
    deg = p_ref[0, :, 0] + p_ref[1, :, 0] + 1.0
    dinv = lax.rsqrt(deg)
    xw = lax.dot_general(
        x_ref[...], w_ref[...], (((1,), (1,)), ((), ())),
        preferred_element_type=jnp.float32,
    )
    y3_ref[...] = xw * dinv[:, None]
    dinv_ref[...] = dinv[:, None]


def _mm_call(x, W, partials):
    return pl.pallas_call(
        _mm_body,
        grid=(NRB, 2),
        in_specs=[
            pl.BlockSpec((R, D), lambda i, c: (i, 0)),
            pl.BlockSpec((H, D), lambda i, c: (c, 0)),
            pl.BlockSpec((2, R, H), lambda i, c: (0, i, 0)),
        ],
        out_specs=[
            pl.BlockSpec((R, H), lambda i, c: (c * NRB + i, 0)),
            pl.BlockSpec((R, 1), lambda i, c: (i, 0)),
        ],
        out_shape=[
            jax.ShapeDtypeStruct((2 * N, H), jnp.float32),
            jax.ShapeDtypeStruct((N, 1), jnp.float32),
        ],
    )(x, W, partials)


@functools.partial(
    pl.kernel,
    out_type=jax.ShapeDtypeStruct((2 * N, H), jnp.float32),
    mesh=_mesh,
    scratch_types=[
        pltpu.VMEM((PASS, K), jnp.int32),
        pltpu.VMEM((PASS, K), jnp.int32),
        pltpu.VMEM((K, H), jnp.float32),
        pltpu.VMEM((K, H), jnp.float32),
        pltpu.VMEM_SHARED((NT, H), jnp.float32),
        pltpu.SemaphoreType.DMA,
        pltpu.SemaphoreType.DMA,
        pltpu.SemaphoreType.DMA,
        pltpu.SemaphoreType.DMA,
    ],
)
def _agg_kernel(gidx_hbm, col3_hbm, y3_hbm, out_hbm,
                gixv, colv, bufa, bufb, acc, sema, sema2, semb, semb2):
    c = lax.axis_index("c")
    s = lax.axis_index("s")
    r0 = s * RPT

    @pl.when(s < TILES - 1)
    def _():
        pltpu.sync_copy(y3_hbm.at[pl.ds(c * N + r0, RPT)], acc.at[pl.ds(r0, RPT)])

    @pl.when(s == TILES - 1)
    def _():
        pltpu.sync_copy(y3_hbm.at[pl.ds(c * N + r0, LASTV)], acc.at[pl.ds(r0, LASTV)])

    plsc.subcore_barrier()

    HK = K // 2

    def _gather(kk, buf, s1, s2):
        pltpu.async_copy(y3_hbm.at[gixv.at[kk, pl.ds(0, HK)]],
                         buf.at[pl.ds(0, HK)], s1)
        pltpu.async_copy(y3_hbm.at[gixv.at[kk, pl.ds(HK, HK)]],
                         buf.at[pl.ds(HK, HK)], s2)

    def _gwait(kk, buf, s1, s2):
        pltpu.make_async_copy(y3_hbm.at[gixv.at[kk, pl.ds(0, HK)]],
                              buf.at[pl.ds(0, HK)], s1).wait()
        pltpu.make_async_copy(y3_hbm.at[gixv.at[kk, pl.ds(HK, HK)]],
                              buf.at[pl.ds(HK, HK)], s2).wait()

    for p in range(2):
        base = p * PASS
        pltpu.sync_copy(gidx_hbm.at[c, s, pl.ds(base, PASS)], gixv)
        pltpu.sync_copy(col3_hbm.at[s, pl.ds(base, PASS)], colv)
        _gather(0, bufa, sema, sema2)
        _gather(1, bufb, semb, semb2)

        def body(k2, carry):
            k = k2 * 2
            _gwait(k, bufa, sema, sema2)
            pltpu.sync_copy(bufa, acc.at[colv.at[k]], add=True)

            @pl.when(k + 2 < PASS)
            def _():
                _gather(k + 2, bufa, sema, sema2)

            _gwait(k + 1, bufb, semb, semb2)
            pltpu.sync_copy(bufb, acc.at[colv.at[k + 1]], add=True)

            @pl.when(k + 3 < PASS)
            def _():
                _gather(k + 3, bufb, semb, semb2)

            return carry

        lax.fori_loop(0, PASS // 2, body, 0)

    plsc.subcore_barrier()

    @pl.when(s < TILES - 1)
    def _():
        pltpu.sync_copy(acc.at[pl.ds(r0, RPT)], out_hbm.at[pl.ds(c * N + r0, RPT)])

    @pl.when(s == TILES - 1)
    def _():
        pltpu.sync_copy(acc.at[pl.ds(r0, LASTV)], out_hbm.at[pl.ds(c * N + r0, LASTV)])


def _fin_body(al_ref, ar_ref, dinv_ref, b_ref, out_ref):
    acc = jnp.concatenate([al_ref[...], ar_ref[...]], axis=1)
    out_ref[...] = acc * dinv_ref[...] + b_ref[...]


def _fin_call(acc, dinv, b2):
    return pl.pallas_call(
        _fin_body,
        grid=(NRB,),
        in_specs=[
            pl.BlockSpec((R, H), lambda i: (i, 0)),
            pl.BlockSpec((R, H), lambda i: (NRB + i, 0)),
            pl.BlockSpec((R, 1), lambda i: (i, 0)),
            pl.BlockSpec((1, D), lambda i: (0, 0)),
        ],
        out_specs=pl.BlockSpec((R, D), lambda i: (i, 0)),
        out_shape=jax.ShapeDtypeStruct((N, D), jnp.float32),
    )(acc, acc, dinv, b2)


def kernel(x, edge_index, W, b):
    row = edge_index[0]
    col = edge_index[1]
    g = jnp.pad(row.reshape(TILES, EPT), ((0, 0), (0, PAD)),
                constant_values=0).reshape(TILES, CHUNKS, K)
    col3 = jnp.pad(col.reshape(TILES, EPT), ((0, 0), (0, PAD)),
                   constant_values=N).reshape(TILES, CHUNKS, K)
    gidx4 = jnp.stack([g, g + N])
    ones_h = jnp.ones((K, H), jnp.float32)
    zeros_h = jnp.zeros((NT, H), jnp.float32)

    partials = _deg_kernel(col3, ones_h, zeros_h)
    y3, dinv = _mm_call(x, W, partials)
    acc = _agg_kernel(gidx4, col3, y3)
    return _fin_call(acc, dinv, b.reshape(1, D))


# speedup vs baseline: 1.0829x; 1.0829x over previous
import functools

import jax
import jax.numpy as jnp
from jax import lax
from jax.experimental import pallas as pl
from jax.experimental.pallas import tpu as pltpu
from jax.experimental.pallas import tpu_sc as plsc

N = 10000
E = 160000
D = 256
H = 128
NT = 10112
TILES = 16
EPT = E // TILES
K = 128
CHUNKS = 80
PAD = CHUNKS * K - EPT
PASS = CHUNKS // 2
RPT = NT // TILES
LASTV = N - (TILES - 1) * RPT
R = 400
NRB = N // R

_mesh = plsc.VectorSubcoreMesh(core_axis_name="c", subcore_axis_name="s")


@functools.partial(
    pl.kernel,
    out_type=jax.ShapeDtypeStruct((N,), jnp.float32),
    mesh=_mesh,
    scratch_types=[
        pltpu.VMEM((CHUNKS, K), jnp.int32),
        pltpu.VMEM((K,), jnp.float32),
        pltpu.VMEM((640,), jnp.float32),
        pltpu.VMEM_SHARED((NT,), jnp.float32),
    ],
)
def _deg_kernel(col3_hbm, dinv_hbm, colv, onesv, degv, table):
    c = lax.axis_index("c")
    s = lax.axis_index("s")
    r0 = s * RPT

    def fill(i, carry):
        degv[pl.ds(i * 16, 16)] = jnp.zeros((16,), jnp.float32)
        return carry

    lax.fori_loop(0, 640 // 16, fill, 0)

    def ofill(i, carry):
        onesv[pl.ds(i * 16, 16)] = jnp.ones((16,), jnp.float32)
        return carry

    lax.fori_loop(0, K // 16, ofill, 0)
    pltpu.sync_copy(degv.at[pl.ds(0, RPT)], table.at[pl.ds(r0, RPT)])
    pltpu.sync_copy(col3_hbm.at[s], colv)
    plsc.subcore_barrier()

    def body(k, carry):
        pltpu.sync_copy(onesv, table.at[colv.at[k]], add=True)
        return carry

    lax.fori_loop(0, CHUNKS, body, 0)
    plsc.subcore_barrier()
    pltpu.sync_copy(table.at[pl.ds(r0, RPT)], degv.at[pl.ds(0, RPT)])

    def dbody(j, carry):
        d = degv[pl.ds(j * 16, 16)] + 1.0
        h = d * 0.5
        xv = d * 0.0 + 0.002
        for _ in range(22):
            xv = xv * (1.5 - h * xv * xv)
        degv[pl.ds(j * 16, 16)] = xv
        return carry

    lax.fori_loop(0, 640 // 16, dbody, 0)

    @pl.when(jnp.logical_and(c == 0, s < TILES - 1))
    def _():
        pltpu.sync_copy(degv.at[pl.ds(0, RPT)], dinv_hbm.at[pl.ds(r0, RPT)])

    @pl.when(jnp.logical_and(c == 0, s == TILES - 1))
    def _():
        pltpu.sync_copy(degv.at[pl.ds(0, LASTV)], dinv_hbm.at[pl.ds(r0, LASTV)])


def _mm_body(x_ref, w_ref, d_ref, y3_ref):
    xw = lax.dot_general(
        x_ref[...], w_ref[...], (((1,), (1,)), ((), ())),
        preferred_element_type=jnp.float32,
    )
    y3_ref[...] = xw * d_ref[...]


def _mm_call(x, W, dinv1):
    return pl.pallas_call(
        _mm_body,
        grid=(NRB, 2),
        in_specs=[
            pl.BlockSpec((R, D), lambda i, c: (i, 0)),
            pl.BlockSpec((H, D), lambda i, c: (c, 0)),
            pl.BlockSpec((R, 1), lambda i, c: (i, 0)),
        ],
        out_specs=pl.BlockSpec((R, H), lambda i, c: (c * NRB + i, 0)),
        out_shape=jax.ShapeDtypeStruct((2 * N, H), jnp.float32),
    )(x, W, dinv1)


@functools.partial(
    pl.kernel,
    out_type=jax.ShapeDtypeStruct((2 * N, H), jnp.float32),
    mesh=_mesh,
    scratch_types=[
        pltpu.VMEM((PASS, K), jnp.int32),
        pltpu.VMEM((PASS, K), jnp.int32),
        pltpu.VMEM((K, H), jnp.float32),
        pltpu.VMEM((K, H), jnp.float32),
        pltpu.VMEM_SHARED((NT, H), jnp.float32),
        pltpu.SemaphoreType.DMA,
        pltpu.SemaphoreType.DMA,
        pltpu.SemaphoreType.DMA,
        pltpu.SemaphoreType.DMA,
    ],
)
def _agg_kernel(gidx_hbm, col3_hbm, y3_hbm, out_hbm,
                gixv, colv, bufa, bufb, acc, sema, sema2, semb, semb2):
    c = lax.axis_index("c")
    s = lax.axis_index("s")
    r0 = s * RPT

    @pl.when(s < TILES - 1)
    def _():
        pltpu.sync_copy(y3_hbm.at[pl.ds(c * N + r0, RPT)], acc.at[pl.ds(r0, RPT)])

    @pl.when(s == TILES - 1)
    def _():
        pltpu.sync_copy(y3_hbm.at[pl.ds(c * N + r0, LASTV)], acc.at[pl.ds(r0, LASTV)])

    plsc.subcore_barrier()

    HK = K // 2

    def _gather(kk, buf, s1, s2):
        pltpu.async_copy(y3_hbm.at[gixv.at[kk, pl.ds(0, HK)]],
                         buf.at[pl.ds(0, HK)], s1)
        pltpu.async_copy(y3_hbm.at[gixv.at[kk, pl.ds(HK, HK)]],
                         buf.at[pl.ds(HK, HK)], s2)

    def _gwait(kk, buf, s1, s2):
        pltpu.make_async_copy(y3_hbm.at[gixv.at[kk, pl.ds(0, HK)]],
                              buf.at[pl.ds(0, HK)], s1).wait()
        pltpu.make_async_copy(y3_hbm.at[gixv.at[kk, pl.ds(HK, HK)]],
                              buf.at[pl.ds(HK, HK)], s2).wait()

    for p in range(2):
        base = p * PASS
        pltpu.sync_copy(gidx_hbm.at[c, s, pl.ds(base, PASS)], gixv)
        pltpu.sync_copy(col3_hbm.at[s, pl.ds(base, PASS)], colv)
        _gather(0, bufa, sema, sema2)
        _gather(1, bufb, semb, semb2)

        def body(k2, carry):
            k = k2 * 2
            _gwait(k, bufa, sema, sema2)
            pltpu.sync_copy(bufa, acc.at[colv.at[k]], add=True)

            @pl.when(k + 2 < PASS)
            def _():
                _gather(k + 2, bufa, sema, sema2)

            _gwait(k + 1, bufb, semb, semb2)
            pltpu.sync_copy(bufb, acc.at[colv.at[k + 1]], add=True)

            @pl.when(k + 3 < PASS)
            def _():
                _gather(k + 3, bufb, semb, semb2)

            return carry

        lax.fori_loop(0, PASS // 2, body, 0)

    plsc.subcore_barrier()

    @pl.when(s < TILES - 1)
    def _():
        pltpu.sync_copy(acc.at[pl.ds(r0, RPT)], out_hbm.at[pl.ds(c * N + r0, RPT)])

    @pl.when(s == TILES - 1)
    def _():
        pltpu.sync_copy(acc.at[pl.ds(r0, LASTV)], out_hbm.at[pl.ds(c * N + r0, LASTV)])


def _fin_body(al_ref, ar_ref, dinv_ref, b_ref, out_ref):
    acc = jnp.concatenate([al_ref[...], ar_ref[...]], axis=1)
    out_ref[...] = acc * dinv_ref[...] + b_ref[...]


def _fin_call(acc, dinv, b2):
    return pl.pallas_call(
        _fin_body,
        grid=(NRB,),
        in_specs=[
            pl.BlockSpec((R, H), lambda i: (i, 0)),
            pl.BlockSpec((R, H), lambda i: (NRB + i, 0)),
            pl.BlockSpec((R, 1), lambda i: (i, 0)),
            pl.BlockSpec((1, D), lambda i: (0, 0)),
        ],
        out_specs=pl.BlockSpec((R, D), lambda i: (i, 0)),
        out_shape=jax.ShapeDtypeStruct((N, D), jnp.float32),
    )(acc, acc, dinv, b2)


def kernel(x, edge_index, W, b):
    row = edge_index[0]
    col = edge_index[1]
    g = jnp.pad(row.reshape(TILES, EPT), ((0, 0), (0, PAD)),
                constant_values=0).reshape(TILES, CHUNKS, K)
    col3 = jnp.pad(col.reshape(TILES, EPT), ((0, 0), (0, PAD)),
                   constant_values=N).reshape(TILES, CHUNKS, K)
    gidx4 = jnp.stack([g, g + N])

    dinv1 = _deg_kernel(col3).reshape(N, 1)
    y3 = _mm_call(x, W, dinv1)
    acc = _agg_kernel(gidx4, col3, y3)
    return _fin_call(acc, dinv1, b.reshape(1, D))


# speedup vs baseline: 1.0885x; 1.0051x over previous
import functools

import jax
import jax.numpy as jnp
from jax import lax
from jax.experimental import pallas as pl
from jax.experimental.pallas import tpu as pltpu
from jax.experimental.pallas import tpu_sc as plsc

N = 10000
E = 160000
D = 256
H = 128
NT = 10112
TILES = 16
EPT = E // TILES
K = 128
CHUNKS = 80
PAD = CHUNKS * K - EPT
PASS = CHUNKS // 2
RPT = NT // TILES
LASTV = N - (TILES - 1) * RPT
R = 400
NRB = N // R

_mesh = plsc.VectorSubcoreMesh(core_axis_name="c", subcore_axis_name="s")


@functools.partial(
    pl.kernel,
    out_type=jax.ShapeDtypeStruct((N,), jnp.float32),
    mesh=_mesh,
    scratch_types=[
        pltpu.VMEM((CHUNKS, K), jnp.int32),
        pltpu.VMEM((K,), jnp.float32),
        pltpu.VMEM((640,), jnp.float32),
        pltpu.VMEM_SHARED((NT,), jnp.float32),
    ],
)
def _deg_kernel(col3_hbm, dinv_hbm, colv, onesv, degv, table):
    c = lax.axis_index("c")
    s = lax.axis_index("s")
    r0 = s * RPT

    def fill(i, carry):
        degv[pl.ds(i * 16, 16)] = jnp.zeros((16,), jnp.float32)
        return carry

    lax.fori_loop(0, 640 // 16, fill, 0)

    def ofill(i, carry):
        onesv[pl.ds(i * 16, 16)] = jnp.ones((16,), jnp.float32)
        return carry

    lax.fori_loop(0, K // 16, ofill, 0)
    pltpu.sync_copy(degv.at[pl.ds(0, RPT)], table.at[pl.ds(r0, RPT)])
    pltpu.sync_copy(col3_hbm.at[s], colv)
    plsc.subcore_barrier()

    def body(k, carry):
        pltpu.sync_copy(onesv, table.at[colv.at[k]], add=True)
        return carry

    lax.fori_loop(0, CHUNKS, body, 0)
    plsc.subcore_barrier()
    pltpu.sync_copy(table.at[pl.ds(r0, RPT)], degv.at[pl.ds(0, RPT)])

    def dbody(j, carry):
        d = degv[pl.ds(j * 16, 16)] + 1.0
        h = d * 0.5
        xv = d * 0.0 + 0.002
        for _ in range(22):
            xv = xv * (1.5 - h * xv * xv)
        degv[pl.ds(j * 16, 16)] = xv
        return carry

    lax.fori_loop(0, 640 // 16, dbody, 0)

    @pl.when(jnp.logical_and(c == 0, s < TILES - 1))
    def _():
        pltpu.sync_copy(degv.at[pl.ds(0, RPT)], dinv_hbm.at[pl.ds(r0, RPT)])

    @pl.when(jnp.logical_and(c == 0, s == TILES - 1))
    def _():
        pltpu.sync_copy(degv.at[pl.ds(0, LASTV)], dinv_hbm.at[pl.ds(r0, LASTV)])


def _mm_body(x_ref, w_ref, d_ref, y3_ref):
    xw = lax.dot_general(
        x_ref[...], w_ref[...], (((1,), (1,)), ((), ())),
        preferred_element_type=jnp.float32,
    )
    y3_ref[0] = xw * d_ref[...]


def _mm_call(x, W, dinv1):
    return pl.pallas_call(
        _mm_body,
        grid=(NRB, 2),
        in_specs=[
            pl.BlockSpec((R, D), lambda i, c: (i, 0)),
            pl.BlockSpec((H, D), lambda i, c: (c, 0)),
            pl.BlockSpec((R, 1), lambda i, c: (i, 0)),
        ],
        out_specs=pl.BlockSpec((1, R, H), lambda i, c: (c, i, 0)),
        out_shape=jax.ShapeDtypeStruct((2, N, H), jnp.float32),
    )(x, W, dinv1)


@functools.partial(
    pl.kernel,
    out_type=jax.ShapeDtypeStruct((2, N, H), jnp.float32),
    mesh=_mesh,
    scratch_types=[
        pltpu.VMEM((PASS, K), jnp.int32),
        pltpu.VMEM((PASS, K), jnp.int32),
        pltpu.VMEM((K, H), jnp.float32),
        pltpu.VMEM((K, H), jnp.float32),
        pltpu.VMEM_SHARED((NT, H), jnp.float32),
        pltpu.SemaphoreType.DMA,
        pltpu.SemaphoreType.DMA,
    ],
)
def _agg_kernel(g3_hbm, col3_hbm, y3_hbm, out_hbm,
                gixv, colv, bufa, bufb, acc, sema, semb):
    c = lax.axis_index("c")
    s = lax.axis_index("s")
    r0 = s * RPT
    yc = y3_hbm.at[c]

    @pl.when(s < TILES - 1)
    def _():
        pltpu.sync_copy(yc.at[pl.ds(r0, RPT)], acc.at[pl.ds(r0, RPT)])

    @pl.when(s == TILES - 1)
    def _():
        pltpu.sync_copy(yc.at[pl.ds(r0, LASTV)], acc.at[pl.ds(r0, LASTV)])

    for p in range(2):
        base = p * PASS
        pltpu.sync_copy(g3_hbm.at[s, pl.ds(base, PASS)], gixv)
        pltpu.sync_copy(col3_hbm.at[s, pl.ds(base, PASS)], colv)
        pltpu.async_copy(yc.at[gixv.at[0]], bufa, sema)
        pltpu.async_copy(yc.at[gixv.at[1]], bufb, semb)
        if p == 0:
            plsc.subcore_barrier()

        def body(k2, carry):
            k = k2 * 2
            pltpu.make_async_copy(yc.at[gixv.at[k]], bufa, sema).wait()
            pltpu.sync_copy(bufa, acc.at[colv.at[k]], add=True)

            @pl.when(k + 2 < PASS)
            def _():
                pltpu.async_copy(yc.at[gixv.at[k + 2]], bufa, sema)

            pltpu.make_async_copy(yc.at[gixv.at[k + 1]], bufb, semb).wait()
            pltpu.sync_copy(bufb, acc.at[colv.at[k + 1]], add=True)

            @pl.when(k + 3 < PASS)
            def _():
                pltpu.async_copy(yc.at[gixv.at[k + 3]], bufb, semb)

            return carry

        lax.fori_loop(0, PASS // 2, body, 0)

    plsc.subcore_barrier()

    @pl.when(s < TILES - 1)
    def _():
        pltpu.sync_copy(acc.at[pl.ds(r0, RPT)], out_hbm.at[c, pl.ds(r0, RPT)])

    @pl.when(s == TILES - 1)
    def _():
        pltpu.sync_copy(acc.at[pl.ds(r0, LASTV)], out_hbm.at[c, pl.ds(r0, LASTV)])


def _fin_body(al_ref, ar_ref, dinv_ref, b_ref, out_ref):
    acc = jnp.concatenate([al_ref[0], ar_ref[0]], axis=1)
    out_ref[...] = acc * dinv_ref[...] + b_ref[...]


def _fin_call(acc, dinv, b2):
    return pl.pallas_call(
        _fin_body,
        grid=(NRB,),
        in_specs=[
            pl.BlockSpec((1, R, H), lambda i: (0, i, 0)),
            pl.BlockSpec((1, R, H), lambda i: (1, i, 0)),
            pl.BlockSpec((R, 1), lambda i: (i, 0)),
            pl.BlockSpec((1, D), lambda i: (0, 0)),
        ],
        out_specs=pl.BlockSpec((R, D), lambda i: (i, 0)),
        out_shape=jax.ShapeDtypeStruct((N, D), jnp.float32),
    )(acc, acc, dinv, b2)


def kernel(x, edge_index, W, b):
    row = edge_index[0]
    col = edge_index[1]
    g = jnp.pad(row.reshape(TILES, EPT), ((0, 0), (0, PAD)),
                constant_values=0).reshape(TILES, CHUNKS, K)
    col3 = jnp.pad(col.reshape(TILES, EPT), ((0, 0), (0, PAD)),
                   constant_values=N).reshape(TILES, CHUNKS, K)
    dinv1 = _deg_kernel(col3).reshape(N, 1)
    y3 = _mm_call(x, W, dinv1)
    acc = _agg_kernel(g, col3, y3)
    return _fin_call(acc, dinv1, b.reshape(1, D))
